# baseline (speedup 1.0000x reference)
import functools

import jax
import jax.numpy as jnp
from jax import lax
from jax.experimental import pallas as pl
from jax.experimental.pallas import tpu as pltpu
from jax.experimental.pallas import tpu_sc as plsc

N, E, B = 10000, 320000, 16
F = 16

NC, NS, L = 2, 16, 16
NW = NC * NS
EPW = E // NW
BATCH = 80
NBATCH = EPW // BATCH
CH = 2000
NCH = EPW // CH
BPC = CH // BATCH
NPAD = 10240
ZR = NPAD // NS


def _nodes_body(x_ref, ng_ref, Wn1_ref, bn1_ref, Wn2_ref, bn2_ref, Gn2_ref,
                bg2_ref, He_s_ref, He_g_ref, be_h_ref,
                tpad_ref, n2_ref, gn_ref, g_ref):
    xv = x_ref[...]
    n1 = jnp.maximum(
        jnp.dot(xv, Wn1_ref[...], preferred_element_type=jnp.float32)
        + bn1_ref[...], 0.0)
    n2 = jnp.maximum(
        jnp.dot(n1, Wn2_ref[...], preferred_element_type=jnp.float32)
        + bn2_ref[...], 0.0)
    oh = (ng_ref[...] == lax.broadcasted_iota(jnp.int32, (1, B), 1))
    oh = oh.astype(jnp.float32)
    n2c = jnp.concatenate([n2, jnp.ones((N, 1), jnp.float32)], axis=1)
    sums = lax.dot_general(oh, n2c, (((0,), (0,)), ((), ())),
                           preferred_element_type=jnp.float32)
    gmean = sums[:, :16] / jnp.maximum(sums[:, 16:17], 1.0)
    g = jnp.dot(gmean, Gn2_ref[...],
                preferred_element_type=jnp.float32) + bg2_ref[...]
    gn = jnp.dot(oh, g, preferred_element_type=jnp.float32)
    t = (jnp.dot(n2, He_s_ref[...], preferred_element_type=jnp.float32)
         + jnp.dot(gn, He_g_ref[...], preferred_element_type=jnp.float32)
         + be_h_ref[...])
    tpad_ref[...] = jnp.concatenate(
        [t, jnp.zeros((N, 8), jnp.float32)], axis=1)
    n2_ref[...] = n2
    gn_ref[...] = gn
    g_ref[...] = g


_nodes_call = pl.pallas_call(
    _nodes_body,
    out_shape=[
        jax.ShapeDtypeStruct((N, F), jnp.float32),
        jax.ShapeDtypeStruct((N, 16), jnp.float32),
        jax.ShapeDtypeStruct((N, 4), jnp.float32),
        jax.ShapeDtypeStruct((B, 4), jnp.float32),
    ],
)


BL = 8000


def _edges_body(ea_ref, We1_ref, be1_ref, We2_ref, be2_ref, He_e_ref,
                apad_ref):
    e1 = jnp.maximum(
        jnp.dot(ea_ref[...], We1_ref[...],
                preferred_element_type=jnp.float32) + be1_ref[...], 0.0)
    e2 = jnp.maximum(
        jnp.dot(e1, We2_ref[...], preferred_element_type=jnp.float32)
        + be2_ref[...], 0.0)
    a = jnp.dot(e2, He_e_ref[...], preferred_element_type=jnp.float32)
    apad_ref[...] = jnp.concatenate(
        [a, jnp.ones((BL, 1), jnp.float32), jnp.zeros((BL, 7), jnp.float32)],
        axis=1)


_edges_call = pl.pallas_call(
    _edges_body,
    grid=(E // BL,),
    in_specs=[
        pl.BlockSpec((BL, 16), lambda i: (i, 0)),
        pl.BlockSpec((16, 4), lambda i: (0, 0)),
        pl.BlockSpec((1, 4), lambda i: (0, 0)),
        pl.BlockSpec((4, 8), lambda i: (0, 0)),
        pl.BlockSpec((1, 8), lambda i: (0, 0)),
        pl.BlockSpec((8, 8), lambda i: (0, 0)),
    ],
    out_specs=pl.BlockSpec((BL, F), lambda i: (i, 0)),
    out_shape=jax.ShapeDtypeStruct((E, F), jnp.float32),
)


_sc_mesh = plsc.VectorSubcoreMesh(core_axis_name="c", subcore_axis_name="s")


@functools.partial(
    pl.kernel,
    mesh=_sc_mesh,
    compiler_params=pltpu.CompilerParams(use_tc_tiling_on_sc=False),
    out_type=[
        jax.ShapeDtypeStruct((NC, NPAD, F), jnp.float32),
        jax.ShapeDtypeStruct((NC, NPAD, F), jnp.float32),
    ],
    scratch_types=[
        pltpu.VMEM((NBATCH, BATCH), jnp.int32),
        pltpu.VMEM((NBATCH, BATCH), jnp.int32),
        pltpu.VMEM((CH, F), jnp.float32),
        pltpu.VMEM((CH, F), jnp.float32),
        pltpu.VMEM_SHARED((NPAD, F), jnp.float32),
        pltpu.VMEM_SHARED((NPAD, F), jnp.float32),
    ],
)
def _sc_hop(tpad_hbm, apad_hbm, s_hbm, r_hbm, outR_hbm, outS_hbm,
            idxS_v, idxR_v, a_v, t_v, accR_s, accS_s):
    cid = lax.axis_index("c")
    sid = lax.axis_index("s")
    wid = cid * NS + sid

    @pl.loop(0, ZR)
    def _zero(r):
        t_v[r] = jnp.zeros((L,), jnp.float32)

    pltpu.sync_copy(t_v.at[pl.ds(0, ZR)], accR_s.at[pl.ds(sid * ZR, ZR)])
    pltpu.sync_copy(t_v.at[pl.ds(0, ZR)], accS_s.at[pl.ds(sid * ZR, ZR)])

    pltpu.sync_copy(s_hbm.at[wid], idxS_v)
    pltpu.sync_copy(r_hbm.at[wid], idxR_v)
    plsc.subcore_barrier()

    @pl.loop(0, NCH)
    def _chunk(c):
        pltpu.sync_copy(apad_hbm.at[pl.ds(wid * EPW + c * CH, CH)], a_v)

        @pl.loop(0, BPC)
        def _gather(j):
            pltpu.sync_copy(tpad_hbm.at[idxS_v.at[c * BPC + j]],
                            t_v.at[pl.ds(j * BATCH, BATCH)])

        @pl.loop(0, CH)
        def _relu(r):
            a_v[r] = jnp.maximum(a_v[r] + t_v[r], 0.0)

        @pl.loop(0, BPC)
        def _scatter(j):
            src = a_v.at[pl.ds(j * BATCH, BATCH)]
            pltpu.sync_copy(src, accR_s.at[idxR_v.at[c * BPC + j]], add=True)
            pltpu.sync_copy(src, accS_s.at[idxS_v.at[c * BPC + j]], add=True)

    plsc.subcore_barrier()
    pltpu.sync_copy(accR_s.at[pl.ds(sid * ZR, ZR)],
                    outR_hbm.at[cid].at[pl.ds(sid * ZR, ZR)])
    pltpu.sync_copy(accS_s.at[pl.ds(sid * ZR, ZR)],
                    outS_hbm.at[cid].at[pl.ds(sid * ZR, ZR)])


def _post_body(accR_ref, accS_ref, n2_ref, gn_ref, g_ref, ng_ref,
               Hn_n_ref, Hn_in_ref, Hn_g_ref, bn_h_ref,
               Hg_e_ref, Hg_n_ref, Hg_g_ref, bg_h_ref,
               Rn_ref, brn_ref, Rg_ref, brg_ref,
               nodes_ref, glob_ref):
    aR = accR_ref[0, :N, :] + accR_ref[1, :N, :]
    inc = aR[:, :8] / jnp.maximum(aR[:, 8:9], 1.0)
    n3 = jnp.maximum(
        jnp.dot(n2_ref[...], Hn_n_ref[...], preferred_element_type=jnp.float32)
        + jnp.dot(inc, Hn_in_ref[...], preferred_element_type=jnp.float32)
        + jnp.dot(gn_ref[...], Hn_g_ref[...],
                  preferred_element_type=jnp.float32)
        + bn_h_ref[...], 0.0)

    oh = (ng_ref[...] == lax.broadcasted_iota(jnp.int32, (1, B), 1))
    oh = oh.astype(jnp.float32)

    aS = accS_ref[0, :N, :] + accS_ref[1, :N, :]
    egsum = lax.dot_general(oh, aS, (((0,), (0,)), ((), ())),
                            preferred_element_type=jnp.float32)
    egm = egsum[:, :8] / jnp.maximum(egsum[:, 8:9], 1.0)

    n3c = jnp.concatenate([n3, jnp.ones((N, 1), jnp.float32)], axis=1)
    nsum = lax.dot_general(oh, n3c, (((0,), (0,)), ((), ())),
                           preferred_element_type=jnp.float32)
    nm = nsum[:, :16] / jnp.maximum(nsum[:, 16:17], 1.0)

    g2 = jnp.maximum(
        jnp.dot(egm, Hg_e_ref[...], preferred_element_type=jnp.float32)
        + jnp.dot(nm, Hg_n_ref[...], preferred_element_type=jnp.float32)
        + jnp.dot(g_ref[...], Hg_g_ref[...], preferred_element_type=jnp.float32)
        + bg_h_ref[...], 0.0)

    zn = jnp.dot(n3, Rn_ref[...], preferred_element_type=jnp.float32) \
        + brn_ref[...]
    nodes_ref[...] = 1.0 / (1.0 + jnp.exp(-zn))
    zg = jnp.dot(g2, Rg_ref[...], preferred_element_type=jnp.float32) \
        + brg_ref[...]
    glob_ref[...] = 1.0 / (1.0 + jnp.exp(-zg))


_post_call = pl.pallas_call(
    _post_body,
    out_shape=[
        jax.ShapeDtypeStruct((N, 1), jnp.float32),
        jax.ShapeDtypeStruct((B, 1), jnp.float32),
    ],
)


def kernel(x, edge_attr, senders, receivers, node_graph, We1, be1, Wn1, bn1,
           We2, be2, Wn2, bn2, Gn2, bg2, He_e, He_s, He_g, be_h, Hn_n, Hn_in,
           Hn_g, bn_h, Hg_e, Hg_n, Hg_g, bg_h, Rn, brn, Rg, brg):
    ng2 = node_graph.reshape(N, 1)
    tpad, n2, gn, g = _nodes_call(
        x, ng2, Wn1, bn1.reshape(1, -1), Wn2, bn2.reshape(1, -1), Gn2,
        bg2.reshape(1, -1), He_s, He_g, be_h.reshape(1, -1))
    apad = _edges_call(edge_attr, We1, be1.reshape(1, -1), We2,
                       be2.reshape(1, -1), He_e)
    accR, accS = _sc_hop(tpad, apad,
                         senders.reshape(NW, NBATCH, BATCH),
                         receivers.reshape(NW, NBATCH, BATCH))
    nodes, globals_ = _post_call(
        accR, accS, n2, gn, g, ng2, Hn_n, Hn_in, Hn_g, bn_h.reshape(1, -1),
        Hg_e, Hg_n, Hg_g, bg_h.reshape(1, -1), Rn, brn.reshape(1, -1), Rg,
        brg.reshape(1, -1))
    return nodes, globals_


# speedup vs baseline: 1.3133x; 1.3133x over previous
import functools

import jax
import jax.numpy as jnp
from jax import lax
from jax.experimental import pallas as pl
from jax.experimental.pallas import tpu as pltpu
from jax.experimental.pallas import tpu_sc as plsc

N, E, B = 10000, 320000, 16
F = 16

NC, NS, L = 2, 16, 16
NW = NC * NS
EPW = E // NW
BATCH = 80
NBATCH = EPW // BATCH
CH = 2000
NCH = EPW // CH
BPC = CH // BATCH
NPAD = 10240
ZR = NPAD // NS


def _nodes_body(x_ref, ng_ref, Wn1_ref, bn1_ref, Wn2_ref, bn2_ref, Gn2_ref,
                bg2_ref, He_s_ref, He_g_ref, be_h_ref,
                tpad_ref, n2_ref, gn_ref, g_ref):
    xv = x_ref[...]
    n1 = jnp.maximum(
        jnp.dot(xv, Wn1_ref[...], preferred_element_type=jnp.float32)
        + bn1_ref[...], 0.0)
    n2 = jnp.maximum(
        jnp.dot(n1, Wn2_ref[...], preferred_element_type=jnp.float32)
        + bn2_ref[...], 0.0)
    oh = (ng_ref[...] == lax.broadcasted_iota(jnp.int32, (1, B), 1))
    oh = oh.astype(jnp.float32)
    n2c = jnp.concatenate([n2, jnp.ones((N, 1), jnp.float32)], axis=1)
    sums = lax.dot_general(oh, n2c, (((0,), (0,)), ((), ())),
                           preferred_element_type=jnp.float32)
    gmean = sums[:, :16] / jnp.maximum(sums[:, 16:17], 1.0)
    g = jnp.dot(gmean, Gn2_ref[...],
                preferred_element_type=jnp.float32) + bg2_ref[...]
    gn = jnp.dot(oh, g, preferred_element_type=jnp.float32)
    t = (jnp.dot(n2, He_s_ref[...], preferred_element_type=jnp.float32)
         + jnp.dot(gn, He_g_ref[...], preferred_element_type=jnp.float32)
         + be_h_ref[...])
    tpad_ref[...] = jnp.concatenate(
        [t, jnp.zeros((N, 8), jnp.float32)], axis=1)
    n2_ref[...] = n2
    gn_ref[...] = gn
    g_ref[...] = g


_nodes_call = pl.pallas_call(
    _nodes_body,
    out_shape=[
        jax.ShapeDtypeStruct((N, F), jnp.float32),
        jax.ShapeDtypeStruct((N, 16), jnp.float32),
        jax.ShapeDtypeStruct((N, 4), jnp.float32),
        jax.ShapeDtypeStruct((B, 4), jnp.float32),
    ],
)


BL = 8000


def _edges_body(ea_ref, We1_ref, be1_ref, We2_ref, be2_ref, He_e_ref,
                apad_ref):
    e1 = jnp.maximum(
        jnp.dot(ea_ref[...], We1_ref[...],
                preferred_element_type=jnp.float32) + be1_ref[...], 0.0)
    e2 = jnp.maximum(
        jnp.dot(e1, We2_ref[...], preferred_element_type=jnp.float32)
        + be2_ref[...], 0.0)
    a = jnp.dot(e2, He_e_ref[...], preferred_element_type=jnp.float32)
    apad_ref[...] = jnp.concatenate(
        [a, jnp.ones((BL, 1), jnp.float32), jnp.zeros((BL, 7), jnp.float32)],
        axis=1)


_edges_call = pl.pallas_call(
    _edges_body,
    grid=(E // BL,),
    in_specs=[
        pl.BlockSpec((BL, 16), lambda i: (i, 0)),
        pl.BlockSpec((16, 4), lambda i: (0, 0)),
        pl.BlockSpec((1, 4), lambda i: (0, 0)),
        pl.BlockSpec((4, 8), lambda i: (0, 0)),
        pl.BlockSpec((1, 8), lambda i: (0, 0)),
        pl.BlockSpec((8, 8), lambda i: (0, 0)),
    ],
    out_specs=pl.BlockSpec((BL, F), lambda i: (i, 0)),
    out_shape=jax.ShapeDtypeStruct((E, F), jnp.float32),
)


_sc_mesh = plsc.VectorSubcoreMesh(core_axis_name="c", subcore_axis_name="s")


@functools.partial(
    pl.kernel,
    mesh=_sc_mesh,
    compiler_params=pltpu.CompilerParams(use_tc_tiling_on_sc=False),
    out_type=[
        jax.ShapeDtypeStruct((NC, NPAD, F), jnp.float32),
        jax.ShapeDtypeStruct((NC, NPAD, F), jnp.float32),
    ],
    scratch_types=[
        pltpu.VMEM((NBATCH, BATCH), jnp.int32),
        pltpu.VMEM((NBATCH, BATCH), jnp.int32),
        pltpu.VMEM((CH, F), jnp.float32),
        pltpu.VMEM((CH, F), jnp.float32),
        pltpu.VMEM_SHARED((NPAD, F), jnp.float32),
        pltpu.VMEM_SHARED((NPAD, F), jnp.float32),
        pltpu.SemaphoreType.DMA,
        pltpu.SemaphoreType.DMA,
        pltpu.SemaphoreType.DMA,
    ],
)
def _sc_hop(tpad_hbm, apad_hbm, s_hbm, r_hbm, outR_hbm, outS_hbm,
            idxS_v, idxR_v, a_v, t_v, accR_s, accS_s, gsem, ssem, asem):
    cid = lax.axis_index("c")
    sid = lax.axis_index("s")
    wid = cid * NS + sid

    @plsc.parallel_loop(0, ZR, unroll=8)
    def _zero(r):
        t_v[r] = jnp.zeros((L,), jnp.float32)

    z1 = pltpu.async_copy(t_v.at[pl.ds(0, ZR)],
                          accR_s.at[pl.ds(sid * ZR, ZR)], asem)
    z2 = pltpu.async_copy(t_v.at[pl.ds(0, ZR)],
                          accS_s.at[pl.ds(sid * ZR, ZR)], asem)
    i1 = pltpu.async_copy(s_hbm.at[wid], idxS_v, gsem)
    i2 = pltpu.async_copy(r_hbm.at[wid], idxR_v, gsem)
    z1.wait()
    z2.wait()
    i1.wait()
    i2.wait()
    plsc.subcore_barrier()

    @pl.loop(0, NCH)
    def _chunk(c):
        acp = pltpu.async_copy(apad_hbm.at[pl.ds(wid * EPW + c * CH, CH)],
                               a_v, asem)

        @pl.loop(0, BPC)
        def _gfire(j):
            pltpu.async_copy(tpad_hbm.at[idxS_v.at[c * BPC + j]],
                             t_v.at[pl.ds(j * BATCH, BATCH)], gsem)

        acp.wait()

        @pl.loop(0, BPC)
        def _gdrain(j):
            pltpu.make_async_copy(tpad_hbm.at[idxS_v.at[c * BPC + j]],
                                  t_v.at[pl.ds(j * BATCH, BATCH)],
                                  gsem).wait()

        @plsc.parallel_loop(0, CH, unroll=8)
        def _relu(r):
            a_v[r] = jnp.maximum(a_v[r] + t_v[r], 0.0)

        @pl.loop(0, BPC)
        def _sfire(j):
            src = a_v.at[pl.ds(j * BATCH, BATCH)]
            pltpu.async_copy(src, accR_s.at[idxR_v.at[c * BPC + j]], ssem,
                             add=True)
            pltpu.async_copy(src, accS_s.at[idxS_v.at[c * BPC + j]], ssem,
                             add=True)

        @pl.loop(0, BPC)
        def _sdrain(j):
            src = a_v.at[pl.ds(j * BATCH, BATCH)]
            pltpu.make_async_copy(src, accR_s.at[idxR_v.at[c * BPC + j]],
                                  ssem).wait()
            pltpu.make_async_copy(src, accS_s.at[idxS_v.at[c * BPC + j]],
                                  ssem).wait()

    plsc.subcore_barrier()
    w1 = pltpu.async_copy(accR_s.at[pl.ds(sid * ZR, ZR)],
                          outR_hbm.at[cid].at[pl.ds(sid * ZR, ZR)], asem)
    w2 = pltpu.async_copy(accS_s.at[pl.ds(sid * ZR, ZR)],
                          outS_hbm.at[cid].at[pl.ds(sid * ZR, ZR)], asem)
    w1.wait()
    w2.wait()


def _post_body(accR_ref, accS_ref, n2_ref, gn_ref, g_ref, ng_ref,
               Hn_n_ref, Hn_in_ref, Hn_g_ref, bn_h_ref,
               Hg_e_ref, Hg_n_ref, Hg_g_ref, bg_h_ref,
               Rn_ref, brn_ref, Rg_ref, brg_ref,
               nodes_ref, glob_ref):
    aR = accR_ref[0, :N, :] + accR_ref[1, :N, :]
    inc = aR[:, :8] / jnp.maximum(aR[:, 8:9], 1.0)
    n3 = jnp.maximum(
        jnp.dot(n2_ref[...], Hn_n_ref[...], preferred_element_type=jnp.float32)
        + jnp.dot(inc, Hn_in_ref[...], preferred_element_type=jnp.float32)
        + jnp.dot(gn_ref[...], Hn_g_ref[...],
                  preferred_element_type=jnp.float32)
        + bn_h_ref[...], 0.0)

    oh = (ng_ref[...] == lax.broadcasted_iota(jnp.int32, (1, B), 1))
    oh = oh.astype(jnp.float32)

    aS = accS_ref[0, :N, :] + accS_ref[1, :N, :]
    egsum = lax.dot_general(oh, aS, (((0,), (0,)), ((), ())),
                            preferred_element_type=jnp.float32)
    egm = egsum[:, :8] / jnp.maximum(egsum[:, 8:9], 1.0)

    n3c = jnp.concatenate([n3, jnp.ones((N, 1), jnp.float32)], axis=1)
    nsum = lax.dot_general(oh, n3c, (((0,), (0,)), ((), ())),
                           preferred_element_type=jnp.float32)
    nm = nsum[:, :16] / jnp.maximum(nsum[:, 16:17], 1.0)

    g2 = jnp.maximum(
        jnp.dot(egm, Hg_e_ref[...], preferred_element_type=jnp.float32)
        + jnp.dot(nm, Hg_n_ref[...], preferred_element_type=jnp.float32)
        + jnp.dot(g_ref[...], Hg_g_ref[...], preferred_element_type=jnp.float32)
        + bg_h_ref[...], 0.0)

    zn = jnp.dot(n3, Rn_ref[...], preferred_element_type=jnp.float32) \
        + brn_ref[...]
    nodes_ref[...] = 1.0 / (1.0 + jnp.exp(-zn))
    zg = jnp.dot(g2, Rg_ref[...], preferred_element_type=jnp.float32) \
        + brg_ref[...]
    glob_ref[...] = 1.0 / (1.0 + jnp.exp(-zg))


_post_call = pl.pallas_call(
    _post_body,
    out_shape=[
        jax.ShapeDtypeStruct((N, 1), jnp.float32),
        jax.ShapeDtypeStruct((B, 1), jnp.float32),
    ],
)


def kernel(x, edge_attr, senders, receivers, node_graph, We1, be1, Wn1, bn1,
           We2, be2, Wn2, bn2, Gn2, bg2, He_e, He_s, He_g, be_h, Hn_n, Hn_in,
           Hn_g, bn_h, Hg_e, Hg_n, Hg_g, bg_h, Rn, brn, Rg, brg):
    ng2 = node_graph.reshape(N, 1)
    tpad, n2, gn, g = _nodes_call(
        x, ng2, Wn1, bn1.reshape(1, -1), Wn2, bn2.reshape(1, -1), Gn2,
        bg2.reshape(1, -1), He_s, He_g, be_h.reshape(1, -1))
    apad = _edges_call(edge_attr, We1, be1.reshape(1, -1), We2,
                       be2.reshape(1, -1), He_e)
    accR, accS = _sc_hop(tpad, apad,
                         senders.reshape(NW, NBATCH, BATCH),
                         receivers.reshape(NW, NBATCH, BATCH))
    nodes, globals_ = _post_call(
        accR, accS, n2, gn, g, ng2, Hn_n, Hn_in, Hn_g, bn_h.reshape(1, -1),
        Hg_e, Hg_n, Hg_g, bg_h.reshape(1, -1), Rn, brn.reshape(1, -1), Rg,
        brg.reshape(1, -1))
    return nodes, globals_


# speedup vs baseline: 1.9291x; 1.4689x over previous
import functools

import jax
import jax.numpy as jnp
from jax import lax
from jax.experimental import pallas as pl
from jax.experimental.pallas import tpu as pltpu
from jax.experimental.pallas import tpu_sc as plsc

N, E, B = 10000, 320000, 16
F = 16

NC, NS, L = 2, 16, 16
NW = NC * NS
EPW = E // NW
BATCH = 80
NBATCH = EPW // BATCH
CH = 2000
NCH = EPW // CH
BPC = CH // BATCH
NPAD = 10240
ZR = NPAD // NS


def _nodes_body(x_ref, ng_ref, Wn1_ref, bn1_ref, Wn2_ref, bn2_ref, Gn2_ref,
                bg2_ref, He_s_ref, He_g_ref, be_h_ref,
                tpad_ref, n2_ref, gn_ref, g_ref):
    xv = x_ref[...]
    n1 = jnp.maximum(
        jnp.dot(xv, Wn1_ref[...], preferred_element_type=jnp.float32)
        + bn1_ref[...], 0.0)
    n2 = jnp.maximum(
        jnp.dot(n1, Wn2_ref[...], preferred_element_type=jnp.float32)
        + bn2_ref[...], 0.0)
    oh = (ng_ref[...] == lax.broadcasted_iota(jnp.int32, (1, B), 1))
    oh = oh.astype(jnp.float32)
    n2c = jnp.concatenate([n2, jnp.ones((N, 1), jnp.float32)], axis=1)
    sums = lax.dot_general(oh, n2c, (((0,), (0,)), ((), ())),
                           preferred_element_type=jnp.float32)
    gmean = sums[:, :16] / jnp.maximum(sums[:, 16:17], 1.0)
    g = jnp.dot(gmean, Gn2_ref[...],
                preferred_element_type=jnp.float32) + bg2_ref[...]
    gn = jnp.dot(oh, g, preferred_element_type=jnp.float32)
    t = (jnp.dot(n2, He_s_ref[...], preferred_element_type=jnp.float32)
         + jnp.dot(gn, He_g_ref[...], preferred_element_type=jnp.float32)
         + be_h_ref[...])
    tpad_ref[...] = jnp.concatenate(
        [t, jnp.zeros((N, 8), jnp.float32)], axis=1)
    n2_ref[...] = n2
    gn_ref[...] = gn
    g_ref[...] = g


_nodes_call = pl.pallas_call(
    _nodes_body,
    out_shape=[
        jax.ShapeDtypeStruct((N, F), jnp.float32),
        jax.ShapeDtypeStruct((N, 16), jnp.float32),
        jax.ShapeDtypeStruct((N, 4), jnp.float32),
        jax.ShapeDtypeStruct((B, 4), jnp.float32),
    ],
)


BL = 16000


def _edges_body(eaT_ref, We1T_ref, be1_ref, We2T_ref, be2_ref, He_eT_ref,
                apad8_ref):
    e1 = jnp.maximum(
        jnp.dot(We1T_ref[...], eaT_ref[...],
                preferred_element_type=jnp.float32) + be1_ref[...], 0.0)
    e2 = jnp.maximum(
        jnp.dot(We2T_ref[...], e1, preferred_element_type=jnp.float32)
        + be2_ref[...], 0.0)
    aT = jnp.dot(He_eT_ref[...], e2,
                 preferred_element_type=jnp.float32)
    eye8 = (lax.broadcasted_iota(jnp.int32, (8, 8), 0)
            == lax.broadcasted_iota(jnp.int32, (8, 8), 1)).astype(jnp.float32)
    a = lax.dot_general(aT, eye8, (((0,), (0,)), ((), ())),
                        preferred_element_type=jnp.float32)
    y = jnp.concatenate(
        [a, jnp.ones((BL, 1), jnp.float32), jnp.zeros((BL, 7), jnp.float32)],
        axis=1)
    apad8_ref[...] = y


_edges_call = pl.pallas_call(
    _edges_body,
    grid=(E // BL,),
    in_specs=[
        pl.BlockSpec((16, BL), lambda i: (0, i)),
        pl.BlockSpec((4, 16), lambda i: (0, 0)),
        pl.BlockSpec((4, 1), lambda i: (0, 0)),
        pl.BlockSpec((8, 4), lambda i: (0, 0)),
        pl.BlockSpec((8, 1), lambda i: (0, 0)),
        pl.BlockSpec((8, 8), lambda i: (0, 0)),
    ],
    out_specs=pl.BlockSpec((BL, F), lambda i: (i, 0)),
    out_shape=jax.ShapeDtypeStruct((E, F), jnp.float32),
)


_sc_mesh = plsc.VectorSubcoreMesh(core_axis_name="c", subcore_axis_name="s")


@functools.partial(
    pl.kernel,
    mesh=_sc_mesh,
    compiler_params=pltpu.CompilerParams(use_tc_tiling_on_sc=False),
    out_type=[
        jax.ShapeDtypeStruct((NC, NPAD, F), jnp.float32),
        jax.ShapeDtypeStruct((NC, NPAD, F), jnp.float32),
    ],
    scratch_types=[
        pltpu.VMEM((NBATCH, BATCH), jnp.int32),
        pltpu.VMEM((NBATCH, BATCH), jnp.int32),
        pltpu.VMEM((CH, F), jnp.float32),
        pltpu.VMEM((CH, F), jnp.float32),
        pltpu.VMEM_SHARED((NPAD, F), jnp.float32),
        pltpu.VMEM_SHARED((NPAD, F), jnp.float32),
        pltpu.SemaphoreType.DMA,
        pltpu.SemaphoreType.DMA,
        pltpu.SemaphoreType.DMA,
    ],
)
def _sc_hop(tpad_hbm, apad_hbm, s_hbm, r_hbm, outR_hbm, outS_hbm,
            idxS_v, idxR_v, a_v, t_v, accR_s, accS_s, gsem, ssem, asem):
    cid = lax.axis_index("c")
    sid = lax.axis_index("s")
    wid = cid * NS + sid

    @plsc.parallel_loop(0, ZR, unroll=8)
    def _zero(r):
        t_v[r] = jnp.zeros((L,), jnp.float32)

    z1 = pltpu.async_copy(t_v.at[pl.ds(0, ZR)],
                          accR_s.at[pl.ds(sid * ZR, ZR)], asem)
    z2 = pltpu.async_copy(t_v.at[pl.ds(0, ZR)],
                          accS_s.at[pl.ds(sid * ZR, ZR)], asem)
    i1 = pltpu.async_copy(s_hbm.at[wid], idxS_v, gsem)
    i2 = pltpu.async_copy(r_hbm.at[wid], idxR_v, gsem)
    z1.wait()
    z2.wait()
    i1.wait()
    i2.wait()
    plsc.subcore_barrier()

    @pl.loop(0, NCH)
    def _chunk(c):
        acp = pltpu.async_copy(apad_hbm.at[pl.ds(wid * EPW + c * CH, CH)],
                               a_v, asem)

        @pl.loop(0, BPC)
        def _gfire(j):
            pltpu.async_copy(tpad_hbm.at[idxS_v.at[c * BPC + j]],
                             t_v.at[pl.ds(j * BATCH, BATCH)], gsem)

        acp.wait()

        @pl.loop(0, BPC)
        def _gdrain(j):
            pltpu.make_async_copy(tpad_hbm.at[idxS_v.at[c * BPC + j]],
                                  t_v.at[pl.ds(j * BATCH, BATCH)],
                                  gsem).wait()

        @plsc.parallel_loop(0, CH, unroll=8)
        def _relu(r):
            a_v[r] = jnp.maximum(a_v[r] + t_v[r], 0.0)

        @pl.loop(0, BPC)
        def _sfire(j):
            src = a_v.at[pl.ds(j * BATCH, BATCH)]
            pltpu.async_copy(src, accR_s.at[idxR_v.at[c * BPC + j]], ssem,
                             add=True)
            pltpu.async_copy(src, accS_s.at[idxS_v.at[c * BPC + j]], ssem,
                             add=True)

        @pl.loop(0, BPC)
        def _sdrain(j):
            src = a_v.at[pl.ds(j * BATCH, BATCH)]
            pltpu.make_async_copy(src, accR_s.at[idxR_v.at[c * BPC + j]],
                                  ssem).wait()
            pltpu.make_async_copy(src, accS_s.at[idxS_v.at[c * BPC + j]],
                                  ssem).wait()

    plsc.subcore_barrier()
    w1 = pltpu.async_copy(accR_s.at[pl.ds(sid * ZR, ZR)],
                          outR_hbm.at[cid].at[pl.ds(sid * ZR, ZR)], asem)
    w2 = pltpu.async_copy(accS_s.at[pl.ds(sid * ZR, ZR)],
                          outS_hbm.at[cid].at[pl.ds(sid * ZR, ZR)], asem)
    w1.wait()
    w2.wait()


def _post_body(accR_ref, accS_ref, n2_ref, gn_ref, g_ref, ng_ref,
               Hn_n_ref, Hn_in_ref, Hn_g_ref, bn_h_ref,
               Hg_e_ref, Hg_n_ref, Hg_g_ref, bg_h_ref,
               Rn_ref, brn_ref, Rg_ref, brg_ref,
               nodes_ref, glob_ref):
    accR = accR_ref[...]
    accS = accS_ref[...]
    aR = accR[:N, :] + accR[NPAD:NPAD + N, :]
    inc = aR[:, :8] / jnp.maximum(aR[:, 8:9], 1.0)
    n3 = jnp.maximum(
        jnp.dot(n2_ref[...], Hn_n_ref[...], preferred_element_type=jnp.float32)
        + jnp.dot(inc, Hn_in_ref[...], preferred_element_type=jnp.float32)
        + jnp.dot(gn_ref[...], Hn_g_ref[...],
                  preferred_element_type=jnp.float32)
        + bn_h_ref[...], 0.0)

    oh = (ng_ref[...] == lax.broadcasted_iota(jnp.int32, (1, B), 1))
    oh = oh.astype(jnp.float32)

    aS = accS[:N, :] + accS[NPAD:NPAD + N, :]
    egsum = lax.dot_general(oh, aS, (((0,), (0,)), ((), ())),
                            preferred_element_type=jnp.float32)
    egm = egsum[:, :8] / jnp.maximum(egsum[:, 8:9], 1.0)

    n3c = jnp.concatenate([n3, jnp.ones((N, 1), jnp.float32)], axis=1)
    nsum = lax.dot_general(oh, n3c, (((0,), (0,)), ((), ())),
                           preferred_element_type=jnp.float32)
    nm = nsum[:, :16] / jnp.maximum(nsum[:, 16:17], 1.0)

    g2 = jnp.maximum(
        jnp.dot(egm, Hg_e_ref[...], preferred_element_type=jnp.float32)
        + jnp.dot(nm, Hg_n_ref[...], preferred_element_type=jnp.float32)
        + jnp.dot(g_ref[...], Hg_g_ref[...], preferred_element_type=jnp.float32)
        + bg_h_ref[...], 0.0)

    zn = jnp.dot(n3, Rn_ref[...], preferred_element_type=jnp.float32) \
        + brn_ref[...]
    nodes_ref[...] = 1.0 / (1.0 + jnp.exp(-zn))
    zg = jnp.dot(g2, Rg_ref[...], preferred_element_type=jnp.float32) \
        + brg_ref[...]
    glob_ref[...] = 1.0 / (1.0 + jnp.exp(-zg))


_post_call = pl.pallas_call(
    _post_body,
    out_shape=[
        jax.ShapeDtypeStruct((N, 1), jnp.float32),
        jax.ShapeDtypeStruct((B, 1), jnp.float32),
    ],
)


def kernel(x, edge_attr, senders, receivers, node_graph, We1, be1, Wn1, bn1,
           We2, be2, Wn2, bn2, Gn2, bg2, He_e, He_s, He_g, be_h, Hn_n, Hn_in,
           Hn_g, bn_h, Hg_e, Hg_n, Hg_g, bg_h, Rn, brn, Rg, brg):
    ngcol = node_graph.reshape(N, 1)
    tpad8, n2, gn, g = _nodes_call(
        x, ngcol, Wn1, bn1.reshape(1, -1), Wn2, bn2.reshape(1, -1), Gn2,
        bg2.reshape(1, -1), He_s, He_g, be_h.reshape(1, -1))
    apad8 = _edges_call(edge_attr.T, We1.T, be1.reshape(-1, 1), We2.T,
                        be2.reshape(-1, 1), He_e.T)
    accR, accS = _sc_hop(tpad8, apad8,
                         senders.reshape(NW, NBATCH, BATCH),
                         receivers.reshape(NW, NBATCH, BATCH))
    nodes, globals_ = _post_call(
        accR.reshape(NC * NPAD, F),
        accS.reshape(NC * NPAD, F),
        n2, gn, g, ngcol, Hn_n, Hn_in, Hn_g, bn_h.reshape(1, -1),
        Hg_e, Hg_n, Hg_g, bg_h.reshape(1, -1), Rn, brn.reshape(1, -1), Rg,
        brg.reshape(1, -1))
    return nodes, globals_


# speedup vs baseline: 1.9505x; 1.0111x over previous
import functools

import jax
import jax.numpy as jnp
from jax import lax
from jax.experimental import pallas as pl
from jax.experimental.pallas import tpu as pltpu
from jax.experimental.pallas import tpu_sc as plsc

N, E, B = 10000, 320000, 16
F = 16

NC, NS, L = 2, 16, 16
NW = NC * NS
EPW = E // NW
BATCH = 80
NBATCH = EPW // BATCH
CH = 2000
NCH = EPW // CH
BPC = CH // BATCH
NPAD = 10240
ZR = NPAD // NS


def _nodes_body(x_ref, ng_ref, Wn1_ref, bn1_ref, Wn2_ref, bn2_ref, Gn2_ref,
                bg2_ref, He_s_ref, He_g_ref, be_h_ref,
                tpad_ref, n2_ref, gn_ref, g_ref):
    xv = x_ref[...]
    n1 = jnp.maximum(
        jnp.dot(xv, Wn1_ref[...], preferred_element_type=jnp.float32)
        + bn1_ref[...], 0.0)
    n2 = jnp.maximum(
        jnp.dot(n1, Wn2_ref[...], preferred_element_type=jnp.float32)
        + bn2_ref[...], 0.0)
    oh = (ng_ref[...] == lax.broadcasted_iota(jnp.int32, (1, B), 1))
    oh = oh.astype(jnp.float32)
    n2c = jnp.concatenate([n2, jnp.ones((N, 1), jnp.float32)], axis=1)
    sums = lax.dot_general(oh, n2c, (((0,), (0,)), ((), ())),
                           preferred_element_type=jnp.float32)
    gmean = sums[:, :16] / jnp.maximum(sums[:, 16:17], 1.0)
    g = jnp.dot(gmean, Gn2_ref[...],
                preferred_element_type=jnp.float32) + bg2_ref[...]
    gn = jnp.dot(oh, g, preferred_element_type=jnp.float32)
    t = (jnp.dot(n2, He_s_ref[...], preferred_element_type=jnp.float32)
         + jnp.dot(gn, He_g_ref[...], preferred_element_type=jnp.float32)
         + be_h_ref[...])
    tpad_ref[...] = jnp.concatenate(
        [t, jnp.zeros((N, 8), jnp.float32)], axis=1)
    n2_ref[...] = n2
    gn_ref[...] = gn
    g_ref[...] = g


_nodes_call = pl.pallas_call(
    _nodes_body,
    out_shape=[
        jax.ShapeDtypeStruct((N, F), jnp.float32),
        jax.ShapeDtypeStruct((N, 16), jnp.float32),
        jax.ShapeDtypeStruct((N, 4), jnp.float32),
        jax.ShapeDtypeStruct((B, 4), jnp.float32),
    ],
)


BL = 32000


def _edges_body(eaT_ref, We1T_ref, be1_ref, We2T_ref, be2_ref, He_eT_ref,
                apad8_ref):
    e1 = jnp.maximum(
        jnp.dot(We1T_ref[...], eaT_ref[...],
                preferred_element_type=jnp.float32) + be1_ref[...], 0.0)
    e2 = jnp.maximum(
        jnp.dot(We2T_ref[...], e1, preferred_element_type=jnp.float32)
        + be2_ref[...], 0.0)
    aT = jnp.dot(He_eT_ref[...], e2,
                 preferred_element_type=jnp.float32)
    eye8 = (lax.broadcasted_iota(jnp.int32, (8, 8), 0)
            == lax.broadcasted_iota(jnp.int32, (8, 8), 1)).astype(jnp.float32)
    a = lax.dot_general(aT, eye8, (((0,), (0,)), ((), ())),
                        preferred_element_type=jnp.float32)
    y = jnp.concatenate(
        [a, jnp.ones((BL, 1), jnp.float32), jnp.zeros((BL, 7), jnp.float32)],
        axis=1)
    apad8_ref[...] = y


_edges_call = pl.pallas_call(
    _edges_body,
    grid=(E // BL,),
    in_specs=[
        pl.BlockSpec((16, BL), lambda i: (0, i)),
        pl.BlockSpec((4, 16), lambda i: (0, 0)),
        pl.BlockSpec((4, 1), lambda i: (0, 0)),
        pl.BlockSpec((8, 4), lambda i: (0, 0)),
        pl.BlockSpec((8, 1), lambda i: (0, 0)),
        pl.BlockSpec((8, 8), lambda i: (0, 0)),
    ],
    out_specs=pl.BlockSpec((BL, F), lambda i: (i, 0)),
    out_shape=jax.ShapeDtypeStruct((E, F), jnp.float32),
)


_sc_mesh = plsc.VectorSubcoreMesh(core_axis_name="c", subcore_axis_name="s")


@functools.partial(
    pl.kernel,
    mesh=_sc_mesh,
    compiler_params=pltpu.CompilerParams(use_tc_tiling_on_sc=False),
    out_type=[
        jax.ShapeDtypeStruct((NC, NPAD, F), jnp.float32),
        jax.ShapeDtypeStruct((NC, NPAD, F), jnp.float32),
    ],
    scratch_types=[
        pltpu.VMEM((NBATCH, BATCH), jnp.int32),
        pltpu.VMEM((NBATCH, BATCH), jnp.int32),
        pltpu.VMEM((CH, F), jnp.float32),
        pltpu.VMEM((CH, F), jnp.float32),
        pltpu.VMEM_SHARED((NPAD, F), jnp.float32),
        pltpu.VMEM_SHARED((NPAD, F), jnp.float32),
        pltpu.SemaphoreType.DMA,
        pltpu.SemaphoreType.DMA,
        pltpu.SemaphoreType.DMA,
    ],
)
def _sc_hop(tpad_hbm, apad_hbm, s_hbm, r_hbm, outR_hbm, outS_hbm,
            idxS_v, idxR_v, a_v, t_v, accR_s, accS_s, gsem, ssem, asem):
    cid = lax.axis_index("c")
    sid = lax.axis_index("s")
    wid = cid * NS + sid

    @plsc.parallel_loop(0, ZR, unroll=8)
    def _zero(r):
        t_v[r] = jnp.zeros((L,), jnp.float32)

    z1 = pltpu.async_copy(t_v.at[pl.ds(0, ZR)],
                          accR_s.at[pl.ds(sid * ZR, ZR)], asem)
    z2 = pltpu.async_copy(t_v.at[pl.ds(0, ZR)],
                          accS_s.at[pl.ds(sid * ZR, ZR)], asem)
    i1 = pltpu.async_copy(s_hbm.at[wid], idxS_v, gsem)
    i2 = pltpu.async_copy(r_hbm.at[wid], idxR_v, gsem)
    z1.wait()
    z2.wait()
    i1.wait()
    i2.wait()
    plsc.subcore_barrier()

    @pl.loop(0, NCH)
    def _chunk(c):
        acp = pltpu.async_copy(apad_hbm.at[pl.ds(wid * EPW + c * CH, CH)],
                               a_v, asem)

        @pl.loop(0, BPC)
        def _gfire(j):
            pltpu.async_copy(tpad_hbm.at[idxS_v.at[c * BPC + j]],
                             t_v.at[pl.ds(j * BATCH, BATCH)], gsem)

        acp.wait()

        @pl.loop(0, BPC)
        def _gdrain(j):
            pltpu.make_async_copy(tpad_hbm.at[idxS_v.at[c * BPC + j]],
                                  t_v.at[pl.ds(j * BATCH, BATCH)],
                                  gsem).wait()

        @plsc.parallel_loop(0, CH, unroll=8)
        def _relu(r):
            a_v[r] = jnp.maximum(a_v[r] + t_v[r], 0.0)

        @pl.loop(0, BPC)
        def _sfire(j):
            src = a_v.at[pl.ds(j * BATCH, BATCH)]
            pltpu.async_copy(src, accR_s.at[idxR_v.at[c * BPC + j]], ssem,
                             add=True)
            pltpu.async_copy(src, accS_s.at[idxS_v.at[c * BPC + j]], ssem,
                             add=True)

        @pl.loop(0, BPC)
        def _sdrain(j):
            src = a_v.at[pl.ds(j * BATCH, BATCH)]
            pltpu.make_async_copy(src, accR_s.at[idxR_v.at[c * BPC + j]],
                                  ssem).wait()
            pltpu.make_async_copy(src, accS_s.at[idxS_v.at[c * BPC + j]],
                                  ssem).wait()

    plsc.subcore_barrier()
    w1 = pltpu.async_copy(accR_s.at[pl.ds(sid * ZR, ZR)],
                          outR_hbm.at[cid].at[pl.ds(sid * ZR, ZR)], asem)
    w2 = pltpu.async_copy(accS_s.at[pl.ds(sid * ZR, ZR)],
                          outS_hbm.at[cid].at[pl.ds(sid * ZR, ZR)], asem)
    w1.wait()
    w2.wait()


def _post_body(accR_ref, accS_ref, n2_ref, gn_ref, g_ref, ng_ref,
               Hn_n_ref, Hn_in_ref, Hn_g_ref, bn_h_ref,
               Hg_e_ref, Hg_n_ref, Hg_g_ref, bg_h_ref,
               Rn_ref, brn_ref, Rg_ref, brg_ref,
               nodes_ref, glob_ref):
    accR = accR_ref[...]
    accS = accS_ref[...]
    aR = accR[:N, :] + accR[NPAD:NPAD + N, :]
    inc = aR[:, :8] / jnp.maximum(aR[:, 8:9], 1.0)
    n3 = jnp.maximum(
        jnp.dot(n2_ref[...], Hn_n_ref[...], preferred_element_type=jnp.float32)
        + jnp.dot(inc, Hn_in_ref[...], preferred_element_type=jnp.float32)
        + jnp.dot(gn_ref[...], Hn_g_ref[...],
                  preferred_element_type=jnp.float32)
        + bn_h_ref[...], 0.0)

    oh = (ng_ref[...] == lax.broadcasted_iota(jnp.int32, (1, B), 1))
    oh = oh.astype(jnp.float32)

    aS = accS[:N, :] + accS[NPAD:NPAD + N, :]
    egsum = lax.dot_general(oh, aS, (((0,), (0,)), ((), ())),
                            preferred_element_type=jnp.float32)
    egm = egsum[:, :8] / jnp.maximum(egsum[:, 8:9], 1.0)

    n3c = jnp.concatenate([n3, jnp.ones((N, 1), jnp.float32)], axis=1)
    nsum = lax.dot_general(oh, n3c, (((0,), (0,)), ((), ())),
                           preferred_element_type=jnp.float32)
    nm = nsum[:, :16] / jnp.maximum(nsum[:, 16:17], 1.0)

    g2 = jnp.maximum(
        jnp.dot(egm, Hg_e_ref[...], preferred_element_type=jnp.float32)
        + jnp.dot(nm, Hg_n_ref[...], preferred_element_type=jnp.float32)
        + jnp.dot(g_ref[...], Hg_g_ref[...], preferred_element_type=jnp.float32)
        + bg_h_ref[...], 0.0)

    zn = jnp.dot(n3, Rn_ref[...], preferred_element_type=jnp.float32) \
        + brn_ref[...]
    nodes_ref[...] = 1.0 / (1.0 + jnp.exp(-zn))
    zg = jnp.dot(g2, Rg_ref[...], preferred_element_type=jnp.float32) \
        + brg_ref[...]
    glob_ref[...] = 1.0 / (1.0 + jnp.exp(-zg))


_post_call = pl.pallas_call(
    _post_body,
    out_shape=[
        jax.ShapeDtypeStruct((N, 1), jnp.float32),
        jax.ShapeDtypeStruct((B, 1), jnp.float32),
    ],
)


def kernel(x, edge_attr, senders, receivers, node_graph, We1, be1, Wn1, bn1,
           We2, be2, Wn2, bn2, Gn2, bg2, He_e, He_s, He_g, be_h, Hn_n, Hn_in,
           Hn_g, bn_h, Hg_e, Hg_n, Hg_g, bg_h, Rn, brn, Rg, brg):
    ngcol = node_graph.reshape(N, 1)
    tpad8, n2, gn, g = _nodes_call(
        x, ngcol, Wn1, bn1.reshape(1, -1), Wn2, bn2.reshape(1, -1), Gn2,
        bg2.reshape(1, -1), He_s, He_g, be_h.reshape(1, -1))
    apad8 = _edges_call(edge_attr.T, We1.T, be1.reshape(-1, 1), We2.T,
                        be2.reshape(-1, 1), He_e.T)
    accR, accS = _sc_hop(tpad8, apad8,
                         senders.reshape(NW, NBATCH, BATCH),
                         receivers.reshape(NW, NBATCH, BATCH))
    nodes, globals_ = _post_call(
        accR.reshape(NC * NPAD, F),
        accS.reshape(NC * NPAD, F),
        n2, gn, g, ngcol, Hn_n, Hn_in, Hn_g, bn_h.reshape(1, -1),
        Hg_e, Hg_n, Hg_g, bg_h.reshape(1, -1), Rn, brn.reshape(1, -1), Rg,
        brg.reshape(1, -1))
    return nodes, globals_
